# baseline (speedup 1.0000x reference)
import functools

import jax
import jax.numpy as jnp
import numpy as np
from jax import lax
from jax.experimental import pallas as pl
from jax.experimental.pallas import tpu as pltpu
from jax.experimental.pallas import tpu_sc as plsc

B, C, X, Y, S = 8, 1024, 256, 128, 4
CB = 1024
NPAD = 512
NEG_INF = np.float32(-np.inf)
TINY = np.float32(np.finfo(np.float32).tiny)

_K0 = np.uint32(0)
_K1 = np.uint32(42)
_KS2 = np.uint32(int(_K0) ^ int(_K1) ^ 0x1BD11BDA)
_ROT_A = (13, 15, 26, 6)
_ROT_B = (17, 29, 16, 24)


def _rotl(x, d):
    return lax.shift_left(x, np.uint32(d)) | lax.shift_right_logical(
        x, np.uint32(32 - d))


def _threefry_rounds(x0, x1):
    ks = (_K0, _K1, _KS2)
    x0 = x0 + ks[0]
    x1 = x1 + ks[1]
    for group, (rots, ka, kb) in enumerate((
            (_ROT_A, ks[1], ks[2]),
            (_ROT_B, ks[2], ks[0]),
            (_ROT_A, ks[0], ks[1]),
            (_ROT_B, ks[1], ks[2]),
            (_ROT_A, ks[2], ks[0]))):
        for r in rots:
            x0 = x0 + x1
            x1 = _rotl(x1, r) ^ x0
        x0 = x0 + ka
        x1 = x1 + kb + np.uint32(group + 1)
    return x0 ^ x1


_R = CB * S
_JB = 128


_NJB = C // _JB


def _index_kernel(jlist_ref, u_ref, mask_c_ref, gidx_ref, bv_scr, bi_scr):
    b = pl.program_id(0)
    blk = pl.program_id(1)
    u_cnt = u_ref[b, 0]

    @pl.when(blk == 0)
    def _init():
        bv_scr[...] = jnp.full((_R, 1), NEG_INF, jnp.float32)
        bi_scr[...] = jnp.zeros((_R, 1), jnp.int32)

    @pl.when(blk * np.int32(_JB) < u_cnt)
    def _compute():
        r_io = lax.broadcasted_iota(jnp.uint32, (_R, _JB), 0)
        la_io = lax.broadcasted_iota(jnp.int32, (_R, _JB), 1)
        base = b.astype(jnp.uint32) * np.uint32(C * S * C)
        row_term = base + r_io * np.uint32(C)

        jrow = jlist_ref[0, blk, :]
        ctr = row_term + jrow.astype(jnp.uint32).reshape(1, _JB)
        bits = _threefry_rounds(jnp.zeros_like(ctr), ctr)

        fb = lax.shift_right_logical(bits, np.uint32(9)) | np.uint32(0x3F800000)
        u = lax.bitcast_convert_type(fb, jnp.float32) - np.float32(1.0)
        u = jnp.maximum(u + TINY, TINY)
        g = -jnp.log(-jnp.log(u))

        valid = (la_io + blk * np.int32(_JB)) < u_cnt
        vals = jnp.where(valid, g, NEG_INF)

        m = jnp.max(vals, axis=-1, keepdims=True)
        jmat = jnp.broadcast_to(jrow.reshape(1, _JB), (_R, _JB))
        bidx = jnp.min(jnp.where(vals == m, jmat, jnp.int32(C)), axis=-1,
                       keepdims=True)
        bv = bv_scr[...]
        better = m > bv
        bv_scr[...] = jnp.where(better, m, bv)
        bi_scr[...] = jnp.where(better, bidx, bi_scr[...])

    @pl.when(blk == np.int32(_NJB - 1))
    def _emit():
        idx = bi_scr[...].reshape(CB, S)
        mask_c = mask_c_ref[0, 0, :].reshape(CB, 1)
        s_io2 = lax.broadcasted_iota(jnp.int32, (CB, S), 1)
        c_io2 = lax.broadcasted_iota(jnp.int32, (CB, S), 0)
        spread = (c_io2 * np.int32(S) + s_io2) % np.int32(NPAD)
        gidx_ref[0] = jnp.where(mask_c > 0, b * np.int32(C) + idx,
                                np.int32(B * C) + spread)


def _compute_gather_indices(mask_ctx):
    mask3 = mask_ctx.reshape(B, 1, C)
    j_iota = jnp.arange(C, dtype=jnp.int32)[None, :]
    sort_key = jnp.where(mask_ctx > 0, j_iota, j_iota + np.int32(C))
    jlist = jnp.argsort(sort_key, axis=1).astype(jnp.int32).reshape(B, _NJB, _JB)
    u_cnt = jnp.sum(mask_ctx, axis=1, dtype=jnp.int32).reshape(B, 1)

    gidx_cs = pl.pallas_call(
        _index_kernel,
        grid=(B, _NJB),
        in_specs=[
            pl.BlockSpec((1, _NJB, _JB), lambda b, blk: (b, 0, 0)),
            pl.BlockSpec(memory_space=pltpu.SMEM, block_shape=(B, 1),
                         index_map=lambda b, blk: (0, 0)),
            pl.BlockSpec((1, 1, C), lambda b, blk: (b, 0, 0)),
        ],
        out_specs=pl.BlockSpec((1, CB, S), lambda b, blk: (b, 0, 0)),
        out_shape=jax.ShapeDtypeStruct((B, C, S), jnp.int32),
        scratch_shapes=[
            pltpu.VMEM((_R, 1), jnp.float32),
            pltpu.VMEM((_R, 1), jnp.int32),
        ],
    )(jlist, u_cnt, mask3)
    return jnp.transpose(gidx_cs, (0, 2, 1))


_NW = 32
_ROWS = B * S * C
_RPW = _ROWS // _NW
_CHUNK = 256


def _sc_gather(x_tab, y_tab, gidx_flat):
    mesh = plsc.VectorSubcoreMesh(core_axis_name="c", subcore_axis_name="s")

    @functools.partial(
        pl.kernel,
        out_type=(
            jax.ShapeDtypeStruct((_ROWS, X), jnp.float32),
            jax.ShapeDtypeStruct((_ROWS, Y), jnp.float32),
        ),
        mesh=mesh,
        scratch_types=[
            pltpu.VMEM((_CHUNK,), jnp.int32),
            pltpu.VMEM((_CHUNK, X), jnp.float32),
            pltpu.VMEM((_CHUNK, Y), jnp.float32),
            pltpu.SemaphoreType.DMA,
            pltpu.SemaphoreType.DMA,
        ],
    )
    def gather_kernel(x_hbm, y_hbm, idx_hbm, ox_hbm, oy_hbm,
                      idx_v, xrows_v, yrows_v, semx, semy):
        wid = lax.axis_index("s") * 2 + lax.axis_index("c")
        base = wid * _RPW
        for k in range(_RPW // _CHUNK):
            off = base + k * _CHUNK
            pltpu.sync_copy(idx_hbm.at[pl.ds(off, _CHUNK)], idx_v)
            cx = pltpu.async_copy(x_hbm.at[idx_v], xrows_v, semx)
            cy = pltpu.async_copy(y_hbm.at[idx_v], yrows_v, semy)
            cx.wait()
            pltpu.sync_copy(xrows_v, ox_hbm.at[pl.ds(off, _CHUNK)])
            cy.wait()
            pltpu.sync_copy(yrows_v, oy_hbm.at[pl.ds(off, _CHUNK)])

    return gather_kernel(x_tab, y_tab, gidx_flat)


def kernel(x_ctx, y_ctx, mask_ctx, num_samples):
    del num_samples
    gidx = _compute_gather_indices(mask_ctx)

    x_tab = jnp.concatenate(
        [x_ctx.reshape(B * C, X), jnp.zeros((NPAD, X), jnp.float32)])
    y_tab = jnp.concatenate(
        [y_ctx.reshape(B * C, Y), jnp.zeros((NPAD, Y), jnp.float32)])

    out_x, out_y = _sc_gather(x_tab, y_tab, gidx.reshape(_ROWS))
    return (out_x.reshape(B, S, C, X), out_y.reshape(B, S, C, Y))


# speedup vs baseline: 1.0191x; 1.0191x over previous
import functools

import jax
import jax.numpy as jnp
import numpy as np
from jax import lax
from jax.experimental import pallas as pl
from jax.experimental.pallas import tpu as pltpu
from jax.experimental.pallas import tpu_sc as plsc

B, C, X, Y, S = 8, 1024, 256, 128, 4
CB = 1024
NPAD = 512
NEG_INF = np.float32(-np.inf)
TINY = np.float32(np.finfo(np.float32).tiny)

_K0 = np.uint32(0)
_K1 = np.uint32(42)
_KS2 = np.uint32(int(_K0) ^ int(_K1) ^ 0x1BD11BDA)
_ROT_A = (13, 15, 26, 6)
_ROT_B = (17, 29, 16, 24)


def _rotl(x, d):
    return lax.shift_left(x, np.uint32(d)) | lax.shift_right_logical(
        x, np.uint32(32 - d))


def _threefry_rounds(x1):
    ks = (_K0, _K1, _KS2)
    x1 = x1 + ks[1]
    x0 = x1
    x1 = _rotl(x1, _ROT_A[0]) ^ x0
    for r in _ROT_A[1:]:
        x0 = x0 + x1
        x1 = _rotl(x1, r) ^ x0
    x0 = x0 + ks[1]
    x1 = x1 + np.uint32(int(_KS2) + 1)
    for group, (rots, ka, kb) in enumerate((
            (_ROT_B, ks[2], ks[0]),
            (_ROT_A, ks[0], ks[1]),
            (_ROT_B, ks[1], ks[2]),
            (_ROT_A, ks[2], ks[0])), start=2):
        for r in rots:
            x0 = x0 + x1
            x1 = _rotl(x1, r) ^ x0
        x0 = x0 + ka
        x1 = x1 + np.uint32((int(kb) + group) & 0xFFFFFFFF)
    return x0 ^ x1


_R = CB * S
_JB = 128


_NJB = C // _JB


def _index_kernel(jlist_ref, u_ref, mask_c_ref, gidx_ref, bv_scr, bi_scr):
    b = pl.program_id(0)
    blk = pl.program_id(1)
    u_cnt = u_ref[b, 0]

    @pl.when(blk == 0)
    def _init():
        bv_scr[...] = jnp.full((_R, 1), NEG_INF, jnp.float32)
        bi_scr[...] = jnp.zeros((_R, 1), jnp.int32)

    @pl.when(blk * np.int32(_JB) < u_cnt)
    def _compute():
        la_io = lax.broadcasted_iota(jnp.int32, (_R, _JB), 1)
        r_io = lax.broadcasted_iota(jnp.uint32, (_R, _JB), 0)
        base = b.astype(jnp.uint32) * np.uint32(C * S * C)
        row_term = base + r_io * np.uint32(C)

        jrow = jlist_ref[0, blk, :]
        ctr = row_term + jrow.astype(jnp.uint32).reshape(1, _JB)
        bits = _threefry_rounds(ctr)

        fb = lax.shift_right_logical(bits, np.uint32(9)) | np.uint32(0x3F800000)
        u = lax.bitcast_convert_type(fb, jnp.float32) - np.float32(1.0)
        u = jnp.maximum(u + TINY, TINY)
        g = -jnp.log(-jnp.log(u))

        valid = la_io < (u_cnt - blk * np.int32(_JB))
        vals = jnp.where(valid, g, NEG_INF)

        m = jnp.max(vals, axis=-1, keepdims=True)
        jmat = jnp.broadcast_to(jrow.reshape(1, _JB), (_R, _JB))
        bidx = jnp.min(jnp.where(vals == m, jmat, jnp.int32(C)), axis=-1,
                       keepdims=True)
        bv = bv_scr[...]
        better = m > bv
        bv_scr[...] = jnp.where(better, m, bv)
        bi_scr[...] = jnp.where(better, bidx, bi_scr[...])

    @pl.when(blk == np.int32(_NJB - 1))
    def _emit():
        idx = bi_scr[...].reshape(CB, S)
        mask_c = mask_c_ref[0, 0, :].reshape(CB, 1)
        s_io2 = lax.broadcasted_iota(jnp.int32, (CB, S), 1)
        c_io2 = lax.broadcasted_iota(jnp.int32, (CB, S), 0)
        spread = (c_io2 * np.int32(S) + s_io2) % np.int32(NPAD)
        gidx_ref[0] = jnp.where(mask_c > 0, b * np.int32(C) + idx,
                                np.int32(B * C) + spread)


def _compute_gather_indices(mask_ctx):
    mask3 = mask_ctx.reshape(B, 1, C)
    j_iota = jnp.arange(C, dtype=jnp.int32)[None, :]
    sort_key = jnp.where(mask_ctx > 0, j_iota, j_iota + np.int32(C))
    jlist = jnp.argsort(sort_key, axis=1).astype(jnp.int32).reshape(B, _NJB, _JB)
    u_cnt = jnp.sum(mask_ctx, axis=1, dtype=jnp.int32).reshape(B, 1)

    gidx_cs = pl.pallas_call(
        _index_kernel,
        grid=(B, _NJB),
        in_specs=[
            pl.BlockSpec((1, _NJB, _JB), lambda b, blk: (b, 0, 0)),
            pl.BlockSpec(memory_space=pltpu.SMEM, block_shape=(B, 1),
                         index_map=lambda b, blk: (0, 0)),
            pl.BlockSpec((1, 1, C), lambda b, blk: (b, 0, 0)),
        ],
        out_specs=pl.BlockSpec((1, CB, S), lambda b, blk: (b, 0, 0)),
        out_shape=jax.ShapeDtypeStruct((B, C, S), jnp.int32),
        scratch_shapes=[
            pltpu.VMEM((_R, 1), jnp.float32),
            pltpu.VMEM((_R, 1), jnp.int32),
        ],
    )(jlist, u_cnt, mask3)
    return jnp.transpose(gidx_cs, (0, 2, 1))


_NW = 32
_ROWS = B * S * C
_RPW = _ROWS // _NW
_CHUNK = 256


def _sc_gather(x_tab, y_tab, gidx_flat):
    mesh = plsc.VectorSubcoreMesh(core_axis_name="c", subcore_axis_name="s")

    @functools.partial(
        pl.kernel,
        out_type=(
            jax.ShapeDtypeStruct((_ROWS, X), jnp.float32),
            jax.ShapeDtypeStruct((_ROWS, Y), jnp.float32),
        ),
        mesh=mesh,
        scratch_types=[
            pltpu.VMEM((_CHUNK,), jnp.int32),
            pltpu.VMEM((_CHUNK, X), jnp.float32),
            pltpu.VMEM((_CHUNK, Y), jnp.float32),
            pltpu.SemaphoreType.DMA,
            pltpu.SemaphoreType.DMA,
        ],
    )
    def gather_kernel(x_hbm, y_hbm, idx_hbm, ox_hbm, oy_hbm,
                      idx_v, xrows_v, yrows_v, semx, semy):
        wid = lax.axis_index("s") * 2 + lax.axis_index("c")
        base = wid * _RPW
        for k in range(_RPW // _CHUNK):
            off = base + k * _CHUNK
            pltpu.sync_copy(idx_hbm.at[pl.ds(off, _CHUNK)], idx_v)
            cx = pltpu.async_copy(x_hbm.at[idx_v], xrows_v, semx)
            cy = pltpu.async_copy(y_hbm.at[idx_v], yrows_v, semy)
            cx.wait()
            pltpu.sync_copy(xrows_v, ox_hbm.at[pl.ds(off, _CHUNK)])
            cy.wait()
            pltpu.sync_copy(yrows_v, oy_hbm.at[pl.ds(off, _CHUNK)])

    return gather_kernel(x_tab, y_tab, gidx_flat)


def kernel(x_ctx, y_ctx, mask_ctx, num_samples):
    del num_samples
    gidx = _compute_gather_indices(mask_ctx)

    x_tab = jnp.concatenate(
        [x_ctx.reshape(B * C, X), jnp.zeros((NPAD, X), jnp.float32)])
    y_tab = jnp.concatenate(
        [y_ctx.reshape(B * C, Y), jnp.zeros((NPAD, Y), jnp.float32)])

    out_x, out_y = _sc_gather(x_tab, y_tab, gidx.reshape(_ROWS))
    return (out_x.reshape(B, S, C, X), out_y.reshape(B, S, C, Y))


# speedup vs baseline: 1.0370x; 1.0175x over previous
import functools

import jax
import jax.numpy as jnp
import numpy as np
from jax import lax
from jax.experimental import pallas as pl
from jax.experimental.pallas import tpu as pltpu
from jax.experimental.pallas import tpu_sc as plsc

B, C, X, Y, S = 8, 1024, 256, 128, 4
CB = 1024
NPAD = 512
NEG_INF = np.float32(-np.inf)
TINY = np.float32(np.finfo(np.float32).tiny)

_K0 = np.uint32(0)
_K1 = np.uint32(42)
_KS2 = np.uint32(int(_K0) ^ int(_K1) ^ 0x1BD11BDA)
_ROT_A = (13, 15, 26, 6)
_ROT_B = (17, 29, 16, 24)


def _rotl(x, d):
    return lax.shift_left(x, np.uint32(d)) | lax.shift_right_logical(
        x, np.uint32(32 - d))


def _threefry_rounds(x1):
    ks = (_K0, _K1, _KS2)
    x1 = x1 + ks[1]
    x0 = x1
    x1 = _rotl(x1, _ROT_A[0]) ^ x0
    for r in _ROT_A[1:]:
        x0 = x0 + x1
        x1 = _rotl(x1, r) ^ x0
    x0 = x0 + ks[1]
    x1 = x1 + np.uint32(int(_KS2) + 1)
    for group, (rots, ka, kb) in enumerate((
            (_ROT_B, ks[2], ks[0]),
            (_ROT_A, ks[0], ks[1]),
            (_ROT_B, ks[1], ks[2]),
            (_ROT_A, ks[2], ks[0])), start=2):
        for r in rots:
            x0 = x0 + x1
            x1 = _rotl(x1, r) ^ x0
        x0 = x0 + ka
        x1 = x1 + np.uint32((int(kb) + group) & 0xFFFFFFFF)
    return x0 ^ x1


_R = CB * S
_JB = 128


_NJB = C // _JB


def _index_kernel(jlist_ref, u_ref, mask_c_ref, gidx_ref, bv_scr, bi_scr,
                  rt_scr):
    b = pl.program_id(0)
    blk = pl.program_id(1)
    u_cnt = u_ref[b, 0]

    @pl.when(blk == 0)
    def _init():
        bv_scr[...] = jnp.full((_R, 1), NEG_INF, jnp.float32)
        bi_scr[...] = jnp.zeros((_R, 1), jnp.int32)

    @pl.when((b == 0) & (blk == 0))
    def _init_rows():
        r_io = lax.broadcasted_iota(jnp.uint32, (_R, _JB), 0)
        rt_scr[...] = r_io * np.uint32(C)

    @pl.when(blk * np.int32(_JB) < u_cnt)
    def _compute():
        la_io = lax.broadcasted_iota(jnp.int32, (_R, _JB), 1)
        base = b.astype(jnp.uint32) * np.uint32(C * S * C)

        jrow = jlist_ref[0, blk, :]
        ctr = rt_scr[...] + (jrow.astype(jnp.uint32) + base).reshape(1, _JB)
        bits = _threefry_rounds(ctr)

        fb = lax.shift_right_logical(bits, np.uint32(9)) | np.uint32(0x3F800000)
        u = lax.bitcast_convert_type(fb, jnp.float32) - np.float32(1.0)
        u = jnp.maximum(u, TINY)
        g = -jnp.log(-jnp.log(u))

        valid = la_io < (u_cnt - blk * np.int32(_JB))
        vals = jnp.where(valid, g, NEG_INF)

        m = jnp.max(vals, axis=-1, keepdims=True)
        jmat = jnp.broadcast_to(jrow.reshape(1, _JB), (_R, _JB))
        bidx = jnp.min(jnp.where(vals == m, jmat, jnp.int32(C)), axis=-1,
                       keepdims=True)
        bv = bv_scr[...]
        better = m > bv
        bv_scr[...] = jnp.where(better, m, bv)
        bi_scr[...] = jnp.where(better, bidx, bi_scr[...])

    @pl.when(blk == np.int32(_NJB - 1))
    def _emit():
        idx = bi_scr[...].reshape(CB, S)
        mask_c = mask_c_ref[0, 0, :].reshape(CB, 1)
        s_io2 = lax.broadcasted_iota(jnp.int32, (CB, S), 1)
        c_io2 = lax.broadcasted_iota(jnp.int32, (CB, S), 0)
        spread = (c_io2 * np.int32(S) + s_io2) % np.int32(NPAD)
        gidx_ref[0] = jnp.where(mask_c > 0, b * np.int32(C) + idx,
                                np.int32(B * C) + spread)


def _compute_gather_indices(mask_ctx):
    mask3 = mask_ctx.reshape(B, 1, C)
    j_iota = jnp.arange(C, dtype=jnp.int32)[None, :]
    sort_key = jnp.where(mask_ctx > 0, j_iota, j_iota + np.int32(C))
    jlist = jnp.argsort(sort_key, axis=1).astype(jnp.int32).reshape(B, _NJB, _JB)
    u_cnt = jnp.sum(mask_ctx, axis=1, dtype=jnp.int32).reshape(B, 1)

    gidx_cs = pl.pallas_call(
        _index_kernel,
        grid=(B, _NJB),
        in_specs=[
            pl.BlockSpec((1, _NJB, _JB), lambda b, blk: (b, 0, 0)),
            pl.BlockSpec(memory_space=pltpu.SMEM, block_shape=(B, 1),
                         index_map=lambda b, blk: (0, 0)),
            pl.BlockSpec((1, 1, C), lambda b, blk: (b, 0, 0)),
        ],
        out_specs=pl.BlockSpec((1, CB, S), lambda b, blk: (b, 0, 0)),
        out_shape=jax.ShapeDtypeStruct((B, C, S), jnp.int32),
        scratch_shapes=[
            pltpu.VMEM((_R, 1), jnp.float32),
            pltpu.VMEM((_R, 1), jnp.int32),
            pltpu.VMEM((_R, _JB), jnp.uint32),
        ],
    )(jlist, u_cnt, mask3)
    return jnp.transpose(gidx_cs, (0, 2, 1))


_NW = 32
_ROWS = B * S * C
_RPW = _ROWS // _NW
_CHUNK = 256


def _sc_gather(x_tab, y_tab, gidx_flat):
    mesh = plsc.VectorSubcoreMesh(core_axis_name="c", subcore_axis_name="s")

    @functools.partial(
        pl.kernel,
        out_type=(
            jax.ShapeDtypeStruct((_ROWS, X), jnp.float32),
            jax.ShapeDtypeStruct((_ROWS, Y), jnp.float32),
        ),
        mesh=mesh,
        scratch_types=[
            pltpu.VMEM((_CHUNK,), jnp.int32),
            pltpu.VMEM((_CHUNK, X), jnp.float32),
            pltpu.VMEM((_CHUNK, Y), jnp.float32),
            pltpu.SemaphoreType.DMA,
            pltpu.SemaphoreType.DMA,
        ],
    )
    def gather_kernel(x_hbm, y_hbm, idx_hbm, ox_hbm, oy_hbm,
                      idx_v, xrows_v, yrows_v, semx, semy):
        wid = lax.axis_index("s") * 2 + lax.axis_index("c")
        base = wid * _RPW
        for k in range(_RPW // _CHUNK):
            off = base + k * _CHUNK
            pltpu.sync_copy(idx_hbm.at[pl.ds(off, _CHUNK)], idx_v)
            cx = pltpu.async_copy(x_hbm.at[idx_v], xrows_v, semx)
            cy = pltpu.async_copy(y_hbm.at[idx_v], yrows_v, semy)
            cx.wait()
            pltpu.sync_copy(xrows_v, ox_hbm.at[pl.ds(off, _CHUNK)])
            cy.wait()
            pltpu.sync_copy(yrows_v, oy_hbm.at[pl.ds(off, _CHUNK)])

    return gather_kernel(x_tab, y_tab, gidx_flat)


def kernel(x_ctx, y_ctx, mask_ctx, num_samples):
    del num_samples
    gidx = _compute_gather_indices(mask_ctx)

    x_tab = jnp.concatenate(
        [x_ctx.reshape(B * C, X), jnp.zeros((NPAD, X), jnp.float32)])
    y_tab = jnp.concatenate(
        [y_ctx.reshape(B * C, Y), jnp.zeros((NPAD, Y), jnp.float32)])

    out_x, out_y = _sc_gather(x_tab, y_tab, gidx.reshape(_ROWS))
    return (out_x.reshape(B, S, C, X), out_y.reshape(B, S, C, Y))


# speedup vs baseline: 1.0397x; 1.0026x over previous
import functools

import jax
import jax.numpy as jnp
import numpy as np
from jax import lax
from jax.experimental import pallas as pl
from jax.experimental.pallas import tpu as pltpu
from jax.experimental.pallas import tpu_sc as plsc

B, C, X, Y, S = 8, 1024, 256, 128, 4
CB = 1024
NPAD = 512
NEG_INF = np.float32(-np.inf)
TINY = np.float32(np.finfo(np.float32).tiny)

_K0 = np.uint32(0)
_K1 = np.uint32(42)
_KS2 = np.uint32(int(_K0) ^ int(_K1) ^ 0x1BD11BDA)
_ROT_A = (13, 15, 26, 6)
_ROT_B = (17, 29, 16, 24)


def _rotl(x, d):
    return lax.shift_left(x, np.uint32(d)) | lax.shift_right_logical(
        x, np.uint32(32 - d))


def _threefry_rounds(x1):
    ks = (_K0, _K1, _KS2)
    x1 = x1 + ks[1]
    x0 = x1
    x1 = _rotl(x1, _ROT_A[0]) ^ x0
    for r in _ROT_A[1:]:
        x0 = x0 + x1
        x1 = _rotl(x1, r) ^ x0
    x0 = x0 + ks[1]
    x1 = x1 + np.uint32(int(_KS2) + 1)
    for group, (rots, ka, kb) in enumerate((
            (_ROT_B, ks[2], ks[0]),
            (_ROT_A, ks[0], ks[1]),
            (_ROT_B, ks[1], ks[2]),
            (_ROT_A, ks[2], ks[0])), start=2):
        for r in rots:
            x0 = x0 + x1
            x1 = _rotl(x1, r) ^ x0
        x0 = x0 + ka
        x1 = x1 + np.uint32((int(kb) + group) & 0xFFFFFFFF)
    return x0 ^ x1


_R = CB * S
_JB = 128


_NJB = C // _JB


def _index_kernel(jlist_ref, u_ref, mask_c_ref, gidx_ref, bv_scr, bi_scr,
                  rt_scr):
    b = pl.program_id(0)
    blk = pl.program_id(1)
    u_cnt = u_ref[b, 0]

    @pl.when(blk == 0)
    def _init():
        bv_scr[...] = jnp.full((_R, 1), NEG_INF, jnp.float32)
        bi_scr[...] = jnp.zeros((_R, 1), jnp.int32)

    @pl.when((b == 0) & (blk == 0))
    def _init_rows():
        r_io = lax.broadcasted_iota(jnp.uint32, (_R, _JB), 0)
        rt_scr[...] = r_io * np.uint32(C)

    def _block_body(mask_tail):
        base = b.astype(jnp.uint32) * np.uint32(C * S * C)
        jrow = jlist_ref[0, blk, :]
        ctr = rt_scr[...] + (jrow.astype(jnp.uint32) + base).reshape(1, _JB)
        bits = _threefry_rounds(ctr)

        fb = lax.shift_right_logical(bits, np.uint32(9)) | np.uint32(0x3F800000)
        u = lax.bitcast_convert_type(fb, jnp.float32) - np.float32(1.0)
        u = jnp.maximum(u, TINY)
        g = -jnp.log(-jnp.log(u))

        if mask_tail:
            la_io = lax.broadcasted_iota(jnp.int32, (_R, _JB), 1)
            valid = la_io < (u_cnt - blk * np.int32(_JB))
            vals = jnp.where(valid, g, NEG_INF)
        else:
            vals = g

        m = jnp.max(vals, axis=-1, keepdims=True)
        jmat = jnp.broadcast_to(jrow.reshape(1, _JB), (_R, _JB))
        bidx = jnp.min(jnp.where(vals == m, jmat, jnp.int32(C)), axis=-1,
                       keepdims=True)
        bv = bv_scr[...]
        better = m > bv
        bv_scr[...] = jnp.where(better, m, bv)
        bi_scr[...] = jnp.where(better, bidx, bi_scr[...])

    blk_end = (blk + np.int32(1)) * np.int32(_JB)

    @pl.when(blk_end <= u_cnt)
    def _compute_full():
        _block_body(mask_tail=False)

    @pl.when((blk * np.int32(_JB) < u_cnt) & (u_cnt < blk_end))
    def _compute_partial():
        _block_body(mask_tail=True)

    @pl.when(blk == np.int32(_NJB - 1))
    def _emit():
        idx = bi_scr[...].reshape(CB, S)
        mask_c = mask_c_ref[0, 0, :].reshape(CB, 1)
        s_io2 = lax.broadcasted_iota(jnp.int32, (CB, S), 1)
        c_io2 = lax.broadcasted_iota(jnp.int32, (CB, S), 0)
        spread = (c_io2 * np.int32(S) + s_io2) % np.int32(NPAD)
        gidx_ref[0] = jnp.where(mask_c > 0, b * np.int32(C) + idx,
                                np.int32(B * C) + spread)


def _compute_gather_indices(mask_ctx):
    mask3 = mask_ctx.reshape(B, 1, C)
    j_iota = jnp.arange(C, dtype=jnp.int32)[None, :]
    sort_key = jnp.where(mask_ctx > 0, j_iota, j_iota + np.int32(C))
    jlist = jnp.argsort(sort_key, axis=1).astype(jnp.int32).reshape(B, _NJB, _JB)
    u_cnt = jnp.sum(mask_ctx, axis=1, dtype=jnp.int32).reshape(B, 1)

    gidx_cs = pl.pallas_call(
        _index_kernel,
        grid=(B, _NJB),
        in_specs=[
            pl.BlockSpec((1, _NJB, _JB), lambda b, blk: (b, 0, 0)),
            pl.BlockSpec(memory_space=pltpu.SMEM, block_shape=(B, 1),
                         index_map=lambda b, blk: (0, 0)),
            pl.BlockSpec((1, 1, C), lambda b, blk: (b, 0, 0)),
        ],
        out_specs=pl.BlockSpec((1, CB, S), lambda b, blk: (b, 0, 0)),
        out_shape=jax.ShapeDtypeStruct((B, C, S), jnp.int32),
        scratch_shapes=[
            pltpu.VMEM((_R, 1), jnp.float32),
            pltpu.VMEM((_R, 1), jnp.int32),
            pltpu.VMEM((_R, _JB), jnp.uint32),
        ],
    )(jlist, u_cnt, mask3)
    return jnp.transpose(gidx_cs, (0, 2, 1))


_NW = 32
_ROWS = B * S * C
_RPW = _ROWS // _NW
_CHUNK = 256


def _sc_gather(x_tab, y_tab, gidx_flat):
    mesh = plsc.VectorSubcoreMesh(core_axis_name="c", subcore_axis_name="s")

    @functools.partial(
        pl.kernel,
        out_type=(
            jax.ShapeDtypeStruct((_ROWS, X), jnp.float32),
            jax.ShapeDtypeStruct((_ROWS, Y), jnp.float32),
        ),
        mesh=mesh,
        scratch_types=[
            pltpu.VMEM((_CHUNK,), jnp.int32),
            pltpu.VMEM((_CHUNK, X), jnp.float32),
            pltpu.VMEM((_CHUNK, Y), jnp.float32),
            pltpu.SemaphoreType.DMA,
            pltpu.SemaphoreType.DMA,
        ],
    )
    def gather_kernel(x_hbm, y_hbm, idx_hbm, ox_hbm, oy_hbm,
                      idx_v, xrows_v, yrows_v, semx, semy):
        wid = lax.axis_index("s") * 2 + lax.axis_index("c")
        base = wid * _RPW
        for k in range(_RPW // _CHUNK):
            off = base + k * _CHUNK
            pltpu.sync_copy(idx_hbm.at[pl.ds(off, _CHUNK)], idx_v)
            cx = pltpu.async_copy(x_hbm.at[idx_v], xrows_v, semx)
            cy = pltpu.async_copy(y_hbm.at[idx_v], yrows_v, semy)
            cx.wait()
            pltpu.sync_copy(xrows_v, ox_hbm.at[pl.ds(off, _CHUNK)])
            cy.wait()
            pltpu.sync_copy(yrows_v, oy_hbm.at[pl.ds(off, _CHUNK)])

    return gather_kernel(x_tab, y_tab, gidx_flat)


def kernel(x_ctx, y_ctx, mask_ctx, num_samples):
    del num_samples
    gidx = _compute_gather_indices(mask_ctx)

    x_tab = jnp.concatenate(
        [x_ctx.reshape(B * C, X), jnp.zeros((NPAD, X), jnp.float32)])
    y_tab = jnp.concatenate(
        [y_ctx.reshape(B * C, Y), jnp.zeros((NPAD, Y), jnp.float32)])

    out_x, out_y = _sc_gather(x_tab, y_tab, gidx.reshape(_ROWS))
    return (out_x.reshape(B, S, C, X), out_y.reshape(B, S, C, Y))
